# baseline (speedup 1.0000x reference)
import functools
import math

import jax
import jax.numpy as jnp
from jax import lax
from jax.experimental import pallas as pl
from jax.experimental.pallas import tpu as pltpu
from jax.experimental.pallas import tpu_sc as plsc

_NW = 32
_GRID = 16
_NBINS = _GRID ** 3


def _prep_body(cv_ref, idx_ref):
    x = cv_ref[...]
    B, R, L = x.shape
    comp = lax.broadcasted_iota(jnp.int32, (1, 1, L), 2) % 3

    m = jnp.min(x, axis=1, keepdims=True)
    M = jnp.max(x, axis=1, keepdims=True)

    def sel_reduce(red, v, c, init):
        masked = jnp.where(comp == c, v, init)
        return red(masked, axis=2, keepdims=True)

    mn = [sel_reduce(jnp.min, m, c, jnp.inf) for c in range(3)]
    mx = [sel_reduce(jnp.max, M, c, -jnp.inf) for c in range(3)]
    rg = [mx[c] - mn[c] + 1e-6 for c in range(3)]

    mnP = jnp.where(comp == 0, mn[0], jnp.where(comp == 1, mn[1], mn[2]))
    rgP = jnp.where(comp == 0, rg[0], jnp.where(comp == 1, rg[1], rg[2]))
    wP = jnp.where(comp == 0, 256.0, jnp.where(comp == 1, 16.0, 1.0))

    g1 = jnp.float32(_GRID - 1)
    q = ((x - mnP) / rgP * g1).astype(jnp.int32)
    q = jnp.clip(q, 0, _GRID - 1).astype(jnp.float32) * wP

    li = lax.broadcasted_iota(jnp.int32, (L, 128), 0) // 3
    ki = lax.broadcasted_iota(jnp.int32, (L, 128), 1)
    M3 = (li == ki).astype(jnp.float32)
    dn = (((2,), (0,)), ((), ()))
    s = lax.dot_general(q, M3, dn, preferred_element_type=jnp.float32)
    idx_ref[...] = jnp.round(s)


def _featsum_body(feat_ref, acc_ref):
    @pl.when(pl.program_id(0) == 0)
    def _init():
        acc_ref[...] = jnp.zeros_like(acc_ref)

    acc_ref[...] += jnp.sum(feat_ref[...], axis=1)


def _softmax(x):
    m = jnp.max(x, axis=-1, keepdims=True)
    e = jnp.exp(x - m)
    return e / jnp.sum(e, axis=-1, keepdims=True)


def _final_body(hp_ref, sums_ref, w1_ref, b1_ref, w2_ref, b2_ref, gum_ref,
                t_ref, rw_ref, logits_ref, *, B, N, nparts):
    counts = hp_ref[...].reshape(B, nparts, _NBINS).sum(axis=1)
    hist = counts / (jnp.float32(N) + 1e-6)
    dmean = jnp.mean(hist, axis=1, keepdims=True)
    d = hist - dmean
    var = jnp.sum(d * d, axis=1, keepdims=True) / (_NBINS - 1)
    skew = jnp.mean(d * d * d, axis=1, keepdims=True) / (var * jnp.sqrt(var) + 1e-6)
    g_feat = sums_ref[...] / jnp.float32(N)
    lognf = jnp.full((B, 1), (math.log(N) - 8.0) / 4.0, jnp.float32)
    ri = jnp.concatenate([g_feat, lognf, dmean, var, skew], axis=1)
    dn = (((1,), (1,)), ((), ()))
    h = jnp.maximum(
        lax.dot_general(ri, w1_ref[...], dn, preferred_element_type=jnp.float32)
        + b1_ref[...], 0.0)
    logits = (lax.dot_general(h, w2_ref[...], dn, preferred_element_type=jnp.float32)
              + b2_ref[...])
    logits_ref[...] = logits
    rwe = _softmax(logits)
    rwt = _softmax(logits + gum_ref[...])
    rw_ref[...] = jnp.where(t_ref[0, 0] != 0, rwt, rwe)


def _sc_hist_body(idx_hbm, out_hbm, ibuf, hist, *, npts):
    nc = 2
    wid = lax.axis_index("s") * nc + lax.axis_index("c")
    pltpu.sync_copy(idx_hbm.at[pl.ds(wid * npts, npts)], ibuf)

    zeros16 = jnp.zeros((16,), jnp.float32)

    def zero_body(j, carry):
        hist[pl.ds(j * 16, 16)] = zeros16
        return carry

    lax.fori_loop(0, _NBINS // 16, zero_body, 0)

    ones16 = jnp.ones((16,), jnp.float32)

    def body(i, carry):
        idx = ibuf[pl.ds(i * 16, 16)].astype(jnp.int32)
        plsc.addupdate_scatter(hist, [idx], ones16)
        return carry

    lax.fori_loop(0, npts // 16, body, 0)
    pltpu.sync_copy(hist, out_hbm.at[pl.ds(wid * _NBINS, _NBINS)])


def _sc_hist(idx_flat, total):
    npts = total // _NW
    mesh = plsc.VectorSubcoreMesh(core_axis_name="c", subcore_axis_name="s")
    body = functools.partial(_sc_hist_body, npts=npts)
    fn = pl.kernel(
        body,
        mesh=mesh,
        compiler_params=pltpu.CompilerParams(needs_layout_passes=False),
        out_type=jax.ShapeDtypeStruct((_NW * _NBINS,), jnp.float32),
        scratch_types=[
            pltpu.VMEM((npts,), jnp.float32),
            pltpu.VMEM((_NBINS,), jnp.float32),
        ],
    )
    return fn(idx_flat)


def kernel(feat_3d, coords, training, W1, b1, W2, b2):
    B, N, C = feat_3d.shape
    R = N // 128
    cview = coords.reshape(B, R, 384)

    idx = pl.pallas_call(
        _prep_body,
        out_shape=jax.ShapeDtypeStruct((B, R, 128), jnp.float32),
    )(cview)

    hp = _sc_hist(idx.reshape(-1), B * N)
    hp = hp.reshape(_NW, _NBINS)

    chunk = 4096
    sums = pl.pallas_call(
        _featsum_body,
        grid=(N // chunk,),
        in_specs=[pl.BlockSpec((B, chunk, C), lambda i: (0, i, 0))],
        out_specs=pl.BlockSpec((B, C), lambda i: (0, 0)),
        out_shape=jax.ShapeDtypeStruct((B, C), jnp.float32),
    )(feat_3d)

    u = jax.random.uniform(jax.random.key(42), (B, 3), dtype=jnp.float32)
    gumbel = -jnp.log(-jnp.log(u + 1e-10) + 1e-10)
    tflag = jnp.asarray(training, jnp.int32).reshape(1, 1)

    out_sd = jax.ShapeDtypeStruct((B, 3), jnp.float32)
    final = functools.partial(_final_body, B=B, N=N, nparts=_NW // B)
    rw, logits = pl.pallas_call(
        final,
        out_shape=[out_sd, out_sd],
    )(hp, sums, W1, b1.reshape(1, -1), W2, b2.reshape(1, -1), gumbel, tflag)

    return rw, logits


# speedup vs baseline: 2.0281x; 2.0281x over previous
import functools
import math

import jax
import jax.numpy as jnp
from jax import lax
from jax.experimental import pallas as pl
from jax.experimental.pallas import tpu as pltpu
from jax.experimental.pallas import tpu_sc as plsc

_NW = 32
_GRID = 16
_NBINS = _GRID ** 3


def _prep_body(ct_ref, idx_ref, *, B):
    R128 = ct_ref.shape[0] // (3 * B)
    g1 = jnp.float32(_GRID - 1)

    def nrm(v):
        mn = jnp.min(v)
        rg = jnp.max(v) - mn + 1e-6
        q = ((v - mn) / rg * g1).astype(jnp.int32)
        return jnp.clip(q, 0, _GRID - 1).astype(jnp.float32)

    for b in range(B):
        base = b * 3 * R128
        x = ct_ref[pl.ds(base, R128), :]
        y = ct_ref[pl.ds(base + R128, R128), :]
        z = ct_ref[pl.ds(base + 2 * R128, R128), :]
        s = nrm(x) * 256.0 + nrm(y) * 16.0 + nrm(z)
        idx_ref[pl.ds(b * R128, R128), :] = s


def _featsum_body(feat_ref, acc_ref):
    @pl.when(pl.program_id(0) == 0)
    def _init():
        acc_ref[...] = jnp.zeros_like(acc_ref)

    acc_ref[...] += jnp.sum(feat_ref[...], axis=1)


def _softmax(x):
    m = jnp.max(x, axis=-1, keepdims=True)
    e = jnp.exp(x - m)
    return e / jnp.sum(e, axis=-1, keepdims=True)


def _final_body(hp_ref, sums_ref, w1_ref, b1_ref, w2_ref, b2_ref, gum_ref,
                t_ref, rw_ref, logits_ref, *, B, N, nparts):
    rows = _NBINS // 128
    hp = hp_ref[...].reshape(B, nparts, rows, 128)
    counts = jnp.sum(hp, axis=1)
    hist = counts / (jnp.float32(N) + 1e-6)
    dmean = jnp.mean(hist, axis=(1, 2), keepdims=True)
    d = hist - dmean
    var = jnp.sum(d * d, axis=(1, 2), keepdims=True) / (_NBINS - 1)
    skew = (jnp.mean(d * d * d, axis=(1, 2), keepdims=True)
            / (var * jnp.sqrt(var) + 1e-6))
    dmean = dmean.reshape(B, 1)
    var = var.reshape(B, 1)
    skew = skew.reshape(B, 1)
    g_feat = sums_ref[...] / jnp.float32(N)
    lognf = jnp.full((B, 1), (math.log(N) - 8.0) / 4.0, jnp.float32)
    ri = jnp.concatenate([g_feat, lognf, dmean, var, skew], axis=1)
    dn = (((1,), (1,)), ((), ()))
    h = jnp.maximum(
        lax.dot_general(ri, w1_ref[...], dn, preferred_element_type=jnp.float32)
        + b1_ref[...], 0.0)
    logits = (lax.dot_general(h, w2_ref[...], dn, preferred_element_type=jnp.float32)
              + b2_ref[...])
    logits_ref[...] = logits
    rwe = _softmax(logits)
    rwt = _softmax(logits + gum_ref[...])
    rw_ref[...] = jnp.where(t_ref[0, 0] != 0, rwt, rwe)


def _sc_hist_body(idx_hbm, out_hbm, ibuf, hist, *, npts):
    nc = 2
    wid = lax.axis_index("s") * nc + lax.axis_index("c")
    pltpu.sync_copy(idx_hbm.at[pl.ds(wid * npts, npts)], ibuf)

    zeros16 = jnp.zeros((16,), jnp.float32)

    def zero_body(j, carry):
        hist[pl.ds(j * 16, 16)] = zeros16
        return carry

    lax.fori_loop(0, _NBINS // 16, zero_body, 0)

    ones16 = jnp.ones((16,), jnp.float32)

    def body(i, carry):
        idx = ibuf[pl.ds(i * 16, 16)].astype(jnp.int32)
        plsc.addupdate_scatter(hist, [idx], ones16)
        return carry

    lax.fori_loop(0, npts // 16, body, 0)
    pltpu.sync_copy(hist, out_hbm.at[pl.ds(wid * _NBINS, _NBINS)])


def _sc_hist(idx_flat, total):
    npts = total // _NW
    mesh = plsc.VectorSubcoreMesh(core_axis_name="c", subcore_axis_name="s")
    body = functools.partial(_sc_hist_body, npts=npts)
    fn = pl.kernel(
        body,
        mesh=mesh,
        compiler_params=pltpu.CompilerParams(needs_layout_passes=False),
        out_type=jax.ShapeDtypeStruct((_NW * _NBINS,), jnp.float32),
        scratch_types=[
            pltpu.VMEM((npts,), jnp.float32),
            pltpu.VMEM((_NBINS,), jnp.float32),
        ],
    )
    return fn(idx_flat)


def kernel(feat_3d, coords, training, W1, b1, W2, b2):
    B, N, C = feat_3d.shape
    R128 = N // 128
    ct2 = jnp.transpose(coords, (0, 2, 1)).reshape(B * 3 * R128, 128)

    idx = pl.pallas_call(
        functools.partial(_prep_body, B=B),
        out_shape=jax.ShapeDtypeStruct((B * R128, 128), jnp.float32),
    )(ct2)

    hp = _sc_hist(idx.reshape(-1), B * N)
    hp2 = hp.reshape(_NW * (_NBINS // 128), 128)

    chunk = 4096
    sums = pl.pallas_call(
        _featsum_body,
        grid=(N // chunk,),
        in_specs=[pl.BlockSpec((B, chunk, C), lambda i: (0, i, 0))],
        out_specs=pl.BlockSpec((B, C), lambda i: (0, 0)),
        out_shape=jax.ShapeDtypeStruct((B, C), jnp.float32),
    )(feat_3d)

    u = jax.random.uniform(jax.random.key(42), (B, 3), dtype=jnp.float32)
    gumbel = -jnp.log(-jnp.log(u + 1e-10) + 1e-10)
    tflag = jnp.asarray(training, jnp.int32).reshape(1, 1)

    out_sd = jax.ShapeDtypeStruct((B, 3), jnp.float32)
    final = functools.partial(_final_body, B=B, N=N, nparts=_NW // B)
    rw, logits = pl.pallas_call(
        final,
        out_shape=[out_sd, out_sd],
    )(hp2, sums, W1, b1.reshape(1, -1), W2, b2.reshape(1, -1), gumbel, tflag)

    return rw, logits
